# baseline (speedup 1.0000x reference)
import functools

import jax
import jax.numpy as jnp
from jax import lax
from jax.experimental import pallas as pl
from jax.experimental.pallas import tpu as pltpu
from jax.experimental.pallas import tpu_sc as plsc

N = 10000
E = 320000
D = 128
HD = D // 2
G = 64
OUT_DIM = 10

NC = 2
NS = 16
CH = 100
NCH = E // NS // CH
RZ = 624

_MESH = plsc.VectorSubcoreMesh(core_axis_name="c", subcore_axis_name="s",
                               num_cores=NC, num_subcores=NS)


@functools.partial(
    pl.kernel,
    out_type=jax.ShapeDtypeStruct((NC, N, 16), jnp.float32),
    mesh=_MESH,
    compiler_params=pltpu.CompilerParams(use_tc_tiling_on_sc=False),
    scratch_types=[
        pltpu.VMEM((NCH, CH), jnp.int32),
        pltpu.VMEM((CH, 16), jnp.float32),
        pltpu.VMEM_SHARED((N, 16), jnp.float32),
    ],
)
def _deg_kernel(dst_r, cnt_hbm, dst_v, ones_v, cnt_sh):
    core = lax.axis_index("c")
    sid = lax.axis_index("s")
    row0 = sid * RZ
    nblk = jnp.where(sid == NS - 1, (N - (NS - 1) * RZ) // 16, RZ // 16)

    zero = jnp.zeros((16,), jnp.float32)

    def zbody(i, _):
        ones_v[i, pl.ds(0, 16)] = zero
        return 0

    lax.fori_loop(0, 16, zbody, 0)

    def zdma(t, _):
        pltpu.sync_copy(ones_v.at[pl.ds(0, 16)],
                        cnt_sh.at[pl.ds(row0 + t * 16, 16)])
        return 0

    lax.fori_loop(0, nblk, zdma, 0)

    one = jnp.ones((16,), jnp.float32)

    def obody(i, _):
        ones_v[i, pl.ds(0, 16)] = one
        return 0

    lax.fori_loop(0, CH, obody, 0)

    pltpu.sync_copy(dst_r.at[sid], dst_v)
    plsc.subcore_barrier()

    j0 = core * (NCH // 2)

    def ebody(j, _):
        pltpu.sync_copy(ones_v, cnt_sh.at[dst_v.at[j0 + j]], add=True)
        return 0

    lax.fori_loop(0, NCH // 2, ebody, 0)

    plsc.subcore_barrier()

    def wdma(t, _):
        off = row0 + t * 16
        pltpu.sync_copy(cnt_sh.at[pl.ds(off, 16)],
                        cnt_hbm.at[core, pl.ds(off, 16)])
        return 0

    lax.fori_loop(0, nblk, wdma, 0)


@functools.partial(
    pl.kernel,
    out_type=jax.ShapeDtypeStruct((NC, N, HD), jnp.float32),
    mesh=_MESH,
    compiler_params=pltpu.CompilerParams(use_tc_tiling_on_sc=False),
    scratch_types=[
        pltpu.VMEM((NCH, CH), jnp.int32),
        pltpu.VMEM((NCH, CH), jnp.int32),
        pltpu.VMEM((2, CH, HD), jnp.float32),
        pltpu.VMEM_SHARED((N, HD), jnp.float32),
        pltpu.SemaphoreType.DMA,
        pltpu.SemaphoreType.DMA,
        pltpu.SemaphoreType.DMA,
        pltpu.SemaphoreType.DMA,
    ],
)
def _edge_kernel(y_hbm, src_r, dst_r, z_hbm, src_v, dst_v, rows_v, z_sh,
                 gs0, gs1, ss0, ss1):
    core = lax.axis_index("c")
    sid = lax.axis_index("s")
    row0 = sid * RZ
    nblk = jnp.where(sid == NS - 1, (N - (NS - 1) * RZ) // 16, RZ // 16)

    zero = jnp.zeros((16,), jnp.float32)

    def zv(i, _):
        rows_v[0, i // (HD // 16), pl.ds((i % (HD // 16)) * 16, 16)] = zero
        return 0

    lax.fori_loop(0, 16 * (HD // 16), zv, 0)

    def zdma(t, _):
        pltpu.sync_copy(rows_v.at[0, pl.ds(0, 16)],
                        z_sh.at[pl.ds(row0 + t * 16, 16)])
        return 0

    lax.fori_loop(0, nblk, zdma, 0)

    pltpu.sync_copy(src_r.at[sid], src_v)
    pltpu.sync_copy(dst_r.at[sid], dst_v)
    plsc.subcore_barrier()

    ytab = y_hbm.at[core]

    pltpu.async_copy(ytab.at[src_v.at[0]], rows_v.at[0], gs0)

    def ebody(t, _):
        j0 = 2 * t
        j1 = j0 + 1
        pltpu.make_async_copy(ytab.at[src_v.at[j0]], rows_v.at[0],
                              gs0).wait()
        pltpu.async_copy(rows_v.at[0], z_sh.at[dst_v.at[j0]], ss0, add=True)

        @pl.when(t > 0)
        def _():
            pltpu.make_async_copy(rows_v.at[1], z_sh.at[dst_v.at[j0 - 1]],
                                  ss1).wait()

        pltpu.async_copy(ytab.at[src_v.at[j1]], rows_v.at[1], gs1)
        pltpu.make_async_copy(ytab.at[src_v.at[j1]], rows_v.at[1],
                              gs1).wait()
        pltpu.async_copy(rows_v.at[1], z_sh.at[dst_v.at[j1]], ss1, add=True)
        pltpu.make_async_copy(rows_v.at[0], z_sh.at[dst_v.at[j0]],
                              ss0).wait()

        @pl.when(j1 + 1 < NCH)
        def _():
            pltpu.async_copy(ytab.at[src_v.at[j1 + 1]], rows_v.at[0], gs0)

        return 0

    lax.fori_loop(0, NCH // 2, ebody, 0)
    pltpu.make_async_copy(rows_v.at[1], z_sh.at[dst_v.at[NCH - 1]],
                          ss1).wait()

    plsc.subcore_barrier()

    def wdma(t, _):
        off = row0 + t * 16
        pltpu.sync_copy(z_sh.at[pl.ds(off, 16)],
                        z_hbm.at[core, pl.ds(off, 16)])
        return 0

    lax.fori_loop(0, nblk, wdma, 0)


BLK = 2000
NG = N // BLK


def _mm_body(x_ref, w_ref, o_ref):
    o_ref[...] = jnp.dot(x_ref[...], w_ref[...],
                         preferred_element_type=jnp.float32)


def _scale_body(xw_ref, cnt_ref, y_ref):
    s = jnp.sum(jnp.sum(cnt_ref[...], axis=2), axis=0) * 0.0625
    dinv = lax.rsqrt(s + 1.0)
    y = xw_ref[...] * dinv[:, None]
    y_ref[0] = y[:, :HD]
    y_ref[1] = y[:, HD:]


def _tc2_body(z_ref, y1_ref, cnt_ref, b1_ref, w2_ref, y2_ref):
    s = jnp.sum(jnp.sum(cnt_ref[...], axis=2), axis=0) * 0.0625
    dinv = lax.rsqrt(s + 1.0)
    zy = jnp.concatenate([z_ref[0] + y1_ref[0], z_ref[1] + y1_ref[1]], axis=1)
    agg = zy * dinv[:, None] + b1_ref[...]
    h = jnp.maximum(agg, 0.0)
    y2 = jnp.dot(h, w2_ref[...],
                 preferred_element_type=jnp.float32) * dinv[:, None]
    y2_ref[0] = y2[:, :HD]
    y2_ref[1] = y2[:, HD:]


def _tc3_body(z_ref, y2_ref, cnt_ref, b2_ref, batch_ref, wout_ref, bout_ref,
              o_ref, accs, accc):
    i = pl.program_id(0)

    @pl.when(i == 0)
    def _():
        accs[...] = jnp.zeros((G, D), jnp.float32)
        accc[...] = jnp.zeros((G, D), jnp.float32)

    s = jnp.sum(jnp.sum(cnt_ref[...], axis=2), axis=0) * 0.0625
    dinv = lax.rsqrt(s + 1.0)
    zy = jnp.concatenate([z_ref[0] + y2_ref[0], z_ref[1] + y2_ref[1]], axis=1)
    h2 = zy * dinv[:, None] + b2_ref[...]

    bt = batch_ref[0]
    gids = lax.broadcasted_iota(jnp.int32, (G, 1), 0)
    ohT = (gids == bt).astype(jnp.float32)
    accs[...] += jnp.dot(ohT, h2, preferred_element_type=jnp.float32)
    accc[...] += jnp.sum(ohT, axis=1, keepdims=True)

    @pl.when(i == NG - 1)
    def _():
        counts = accc[:, 0:1]
        pooled = accs[...] / jnp.maximum(counts, 1.0)
        logits = jnp.dot(pooled, wout_ref[...],
                         preferred_element_type=jnp.float32) + bout_ref[...]
        lane = lax.broadcasted_iota(jnp.int32, (G, D), 1)
        valid = lane < OUT_DIM
        lm = jnp.where(valid, logits, -1e30)
        m = jnp.max(lm, axis=1, keepdims=True)
        e = jnp.where(valid, jnp.exp(logits - m), 0.0)
        ssum = jnp.sum(e, axis=1, keepdims=True)
        o_ref[...] = (logits - m) - jnp.log(ssum)


def _tc_mm(x, W1):
    return pl.pallas_call(
        _mm_body,
        grid=(NG,),
        in_specs=[
            pl.BlockSpec((BLK, D), lambda i: (i, 0)),
            pl.BlockSpec((D, D), lambda i: (0, 0)),
        ],
        out_specs=pl.BlockSpec((BLK, D), lambda i: (i, 0)),
        out_shape=jax.ShapeDtypeStruct((N, D), jnp.float32),
    )(x, W1)


def _tc_scale(xw, cnt):
    return pl.pallas_call(
        _scale_body,
        grid=(NG,),
        in_specs=[
            pl.BlockSpec((BLK, D), lambda i: (i, 0)),
            pl.BlockSpec((NC, BLK, 16), lambda i: (0, i, 0)),
        ],
        out_specs=pl.BlockSpec((NC, BLK, HD), lambda i: (0, i, 0)),
        out_shape=jax.ShapeDtypeStruct((NC, N, HD), jnp.float32),
    )(xw, cnt)


def _tc2(z, y1, cnt, b1, W2):
    return pl.pallas_call(
        _tc2_body,
        grid=(NG,),
        in_specs=[
            pl.BlockSpec((NC, BLK, HD), lambda i: (0, i, 0)),
            pl.BlockSpec((NC, BLK, HD), lambda i: (0, i, 0)),
            pl.BlockSpec((NC, BLK, 16), lambda i: (0, i, 0)),
            pl.BlockSpec((1, D), lambda i: (0, 0)),
            pl.BlockSpec((D, D), lambda i: (0, 0)),
        ],
        out_specs=pl.BlockSpec((NC, BLK, HD), lambda i: (0, i, 0)),
        out_shape=jax.ShapeDtypeStruct((NC, N, HD), jnp.float32),
    )(z, y1, cnt, b1, W2)


def _tc3(z, y2, cnt, b2, batch3, Wout_p, bout_p):
    return pl.pallas_call(
        _tc3_body,
        grid=(NG,),
        in_specs=[
            pl.BlockSpec((NC, BLK, HD), lambda i: (0, i, 0)),
            pl.BlockSpec((NC, BLK, HD), lambda i: (0, i, 0)),
            pl.BlockSpec((NC, BLK, 16), lambda i: (0, i, 0)),
            pl.BlockSpec((1, D), lambda i: (0, 0)),
            pl.BlockSpec((1, 1, BLK), lambda i: (i, 0, 0)),
            pl.BlockSpec((D, D), lambda i: (0, 0)),
            pl.BlockSpec((1, D), lambda i: (0, 0)),
        ],
        out_specs=pl.BlockSpec((G, D), lambda i: (0, 0)),
        out_shape=jax.ShapeDtypeStruct((G, D), jnp.float32),
        scratch_shapes=[
            pltpu.VMEM((G, D), jnp.float32),
            pltpu.VMEM((G, D), jnp.float32),
        ],
    )(z, y2, cnt, b2, batch3, Wout_p, bout_p)


def kernel(x, edge_index, batch, W1, b1, W2, b2, W_out, b_out):
    src_r = edge_index[0].astype(jnp.int32).reshape(NS, NCH, CH)
    dst_r = edge_index[1].astype(jnp.int32).reshape(NS, NCH, CH)
    batch3 = batch.astype(jnp.int32).reshape(NG, 1, BLK)
    b1r = b1.reshape(1, D)
    b2r = b2.reshape(1, D)
    Wout_p = jnp.zeros((D, D), jnp.float32).at[:, :OUT_DIM].set(W_out)
    bout_p = jnp.zeros((1, D), jnp.float32).at[0, :OUT_DIM].set(b_out)

    xw1 = _tc_mm(x, W1)
    cnt = _deg_kernel(dst_r)
    y1 = _tc_scale(xw1, cnt)
    z1 = _edge_kernel(y1, src_r, dst_r)
    y2 = _tc2(z1, y1, cnt, b1r, W2)
    z2 = _edge_kernel(y2, src_r, dst_r)
    out = _tc3(z2, y2, cnt, b2r, batch3, Wout_p, bout_p)
    return out[:G, :OUT_DIM]


# speedup vs baseline: 1.2327x; 1.2327x over previous
import functools

import jax
import jax.numpy as jnp
from jax import lax
from jax.experimental import pallas as pl
from jax.experimental.pallas import tpu as pltpu
from jax.experimental.pallas import tpu_sc as plsc

N = 10000
E = 320000
D = 128
HD = D // 2
G = 64
OUT_DIM = 10

NC = 2
NS = 16
CH = 100
NCH = E // NS // CH
RZ = 624

_MESH = plsc.VectorSubcoreMesh(core_axis_name="c", subcore_axis_name="s",
                               num_cores=NC, num_subcores=NS)


@functools.partial(
    pl.kernel,
    out_type=jax.ShapeDtypeStruct((NC, N, 16), jnp.float32),
    mesh=_MESH,
    compiler_params=pltpu.CompilerParams(use_tc_tiling_on_sc=False),
    scratch_types=[
        pltpu.VMEM((NCH, CH), jnp.int32),
        pltpu.VMEM((CH, 16), jnp.float32),
        pltpu.VMEM_SHARED((N, 16), jnp.float32),
    ],
)
def _deg_kernel(dst_r, cnt_hbm, dst_v, ones_v, cnt_sh):
    core = lax.axis_index("c")
    sid = lax.axis_index("s")
    row0 = sid * RZ
    nblk = jnp.where(sid == NS - 1, (N - (NS - 1) * RZ) // 16, RZ // 16)

    zero = jnp.zeros((16,), jnp.float32)

    def zbody(i, _):
        ones_v[i, pl.ds(0, 16)] = zero
        return 0

    lax.fori_loop(0, 16, zbody, 0)

    def zdma(t, _):
        pltpu.sync_copy(ones_v.at[pl.ds(0, 16)],
                        cnt_sh.at[pl.ds(row0 + t * 16, 16)])
        return 0

    lax.fori_loop(0, nblk, zdma, 0)

    one = jnp.ones((16,), jnp.float32)

    def obody(i, _):
        ones_v[i, pl.ds(0, 16)] = one
        return 0

    lax.fori_loop(0, CH, obody, 0)

    pltpu.sync_copy(dst_r.at[sid], dst_v)
    plsc.subcore_barrier()

    j0 = core * (NCH // 2)

    def ebody(j, _):
        pltpu.sync_copy(ones_v, cnt_sh.at[dst_v.at[j0 + j]], add=True)
        return 0

    lax.fori_loop(0, NCH // 2, ebody, 0)

    plsc.subcore_barrier()

    def wdma(t, _):
        off = row0 + t * 16
        pltpu.sync_copy(cnt_sh.at[pl.ds(off, 16)],
                        cnt_hbm.at[core, pl.ds(off, 16)])
        return 0

    lax.fori_loop(0, nblk, wdma, 0)


@functools.partial(
    pl.kernel,
    out_type=jax.ShapeDtypeStruct((NC, N, HD), jnp.float32),
    mesh=_MESH,
    compiler_params=pltpu.CompilerParams(use_tc_tiling_on_sc=False),
    scratch_types=[
        pltpu.VMEM((NCH, CH), jnp.int32),
        pltpu.VMEM((NCH, CH), jnp.int32),
        pltpu.VMEM((2, CH, HD), jnp.float32),
        pltpu.VMEM_SHARED((N, HD), jnp.float32),
        pltpu.SemaphoreType.DMA,
        pltpu.SemaphoreType.DMA,
        pltpu.SemaphoreType.DMA,
        pltpu.SemaphoreType.DMA,
    ],
)
def _edge_kernel(y_hbm, src_r, dst_r, z_hbm, src_v, dst_v, rows_v, z_sh,
                 gs0, gs1, ss0, ss1):
    core = lax.axis_index("c")
    sid = lax.axis_index("s")
    row0 = sid * RZ
    nblk = jnp.where(sid == NS - 1, (N - (NS - 1) * RZ) // 16, RZ // 16)

    zero = jnp.zeros((16,), jnp.float32)

    def zv(i, _):
        rows_v[0, i // (HD // 16), pl.ds((i % (HD // 16)) * 16, 16)] = zero
        return 0

    lax.fori_loop(0, 16 * (HD // 16), zv, 0)

    def zdma(t, _):
        pltpu.sync_copy(rows_v.at[0, pl.ds(0, 16)],
                        z_sh.at[pl.ds(row0 + t * 16, 16)])
        return 0

    lax.fori_loop(0, nblk, zdma, 0)

    pltpu.sync_copy(src_r.at[sid], src_v)
    pltpu.sync_copy(dst_r.at[sid], dst_v)
    plsc.subcore_barrier()

    ytab = y_hbm.at[core]

    pltpu.async_copy(ytab.at[src_v.at[0]], rows_v.at[0], gs0)

    def ebody(t, _):
        j0 = 2 * t
        j1 = j0 + 1
        pltpu.async_copy(ytab.at[src_v.at[j1]], rows_v.at[1], gs1)
        pltpu.make_async_copy(ytab.at[src_v.at[j0]], rows_v.at[0],
                              gs0).wait()
        pltpu.sync_copy(rows_v.at[0], z_sh.at[dst_v.at[j0]], add=True)

        @pl.when(t < NCH // 2 - 1)
        def _():
            pltpu.async_copy(ytab.at[src_v.at[j0 + 2]], rows_v.at[0], gs0)

        pltpu.make_async_copy(ytab.at[src_v.at[j1]], rows_v.at[1],
                              gs1).wait()
        pltpu.sync_copy(rows_v.at[1], z_sh.at[dst_v.at[j1]], add=True)
        return 0

    lax.fori_loop(0, NCH // 2, ebody, 0)

    plsc.subcore_barrier()

    def wdma(t, _):
        off = row0 + t * 16
        pltpu.sync_copy(z_sh.at[pl.ds(off, 16)],
                        z_hbm.at[core, pl.ds(off, 16)])
        return 0

    lax.fori_loop(0, nblk, wdma, 0)


BLK = 2000
NG = N // BLK


def _mm_body(x_ref, w_ref, o_ref):
    o_ref[...] = jnp.dot(x_ref[...], w_ref[...],
                         preferred_element_type=jnp.float32)


def _scale_body(xw_ref, cnt_ref, y_ref):
    s = jnp.sum(jnp.sum(cnt_ref[...], axis=2), axis=0) * 0.0625
    dinv = lax.rsqrt(s + 1.0)
    y = xw_ref[...] * dinv[:, None]
    y_ref[0] = y[:, :HD]
    y_ref[1] = y[:, HD:]


def _tc2_body(z_ref, y1_ref, cnt_ref, b1_ref, w2_ref, y2_ref):
    s = jnp.sum(jnp.sum(cnt_ref[...], axis=2), axis=0) * 0.0625
    dinv = lax.rsqrt(s + 1.0)
    zy = jnp.concatenate([z_ref[0] + y1_ref[0], z_ref[1] + y1_ref[1]], axis=1)
    agg = zy * dinv[:, None] + b1_ref[...]
    h = jnp.maximum(agg, 0.0)
    y2 = jnp.dot(h, w2_ref[...],
                 preferred_element_type=jnp.float32) * dinv[:, None]
    y2_ref[0] = y2[:, :HD]
    y2_ref[1] = y2[:, HD:]


def _tc3_body(z_ref, y2_ref, cnt_ref, b2_ref, batch_ref, wout_ref, bout_ref,
              o_ref, accs, accc):
    i = pl.program_id(0)

    @pl.when(i == 0)
    def _():
        accs[...] = jnp.zeros((G, D), jnp.float32)
        accc[...] = jnp.zeros((G, D), jnp.float32)

    s = jnp.sum(jnp.sum(cnt_ref[...], axis=2), axis=0) * 0.0625
    dinv = lax.rsqrt(s + 1.0)
    zy = jnp.concatenate([z_ref[0] + y2_ref[0], z_ref[1] + y2_ref[1]], axis=1)
    h2 = zy * dinv[:, None] + b2_ref[...]

    bt = batch_ref[0]
    gids = lax.broadcasted_iota(jnp.int32, (G, 1), 0)
    ohT = (gids == bt).astype(jnp.float32)
    accs[...] += jnp.dot(ohT, h2, preferred_element_type=jnp.float32)
    accc[...] += jnp.sum(ohT, axis=1, keepdims=True)

    @pl.when(i == NG - 1)
    def _():
        counts = accc[:, 0:1]
        pooled = accs[...] / jnp.maximum(counts, 1.0)
        logits = jnp.dot(pooled, wout_ref[...],
                         preferred_element_type=jnp.float32) + bout_ref[...]
        lane = lax.broadcasted_iota(jnp.int32, (G, D), 1)
        valid = lane < OUT_DIM
        lm = jnp.where(valid, logits, -1e30)
        m = jnp.max(lm, axis=1, keepdims=True)
        e = jnp.where(valid, jnp.exp(logits - m), 0.0)
        ssum = jnp.sum(e, axis=1, keepdims=True)
        o_ref[...] = (logits - m) - jnp.log(ssum)


def _tc_mm(x, W1):
    return pl.pallas_call(
        _mm_body,
        grid=(NG,),
        in_specs=[
            pl.BlockSpec((BLK, D), lambda i: (i, 0)),
            pl.BlockSpec((D, D), lambda i: (0, 0)),
        ],
        out_specs=pl.BlockSpec((BLK, D), lambda i: (i, 0)),
        out_shape=jax.ShapeDtypeStruct((N, D), jnp.float32),
    )(x, W1)


def _tc_scale(xw, cnt):
    return pl.pallas_call(
        _scale_body,
        grid=(NG,),
        in_specs=[
            pl.BlockSpec((BLK, D), lambda i: (i, 0)),
            pl.BlockSpec((NC, BLK, 16), lambda i: (0, i, 0)),
        ],
        out_specs=pl.BlockSpec((NC, BLK, HD), lambda i: (0, i, 0)),
        out_shape=jax.ShapeDtypeStruct((NC, N, HD), jnp.float32),
    )(xw, cnt)


def _tc2(z, y1, cnt, b1, W2):
    return pl.pallas_call(
        _tc2_body,
        grid=(NG,),
        in_specs=[
            pl.BlockSpec((NC, BLK, HD), lambda i: (0, i, 0)),
            pl.BlockSpec((NC, BLK, HD), lambda i: (0, i, 0)),
            pl.BlockSpec((NC, BLK, 16), lambda i: (0, i, 0)),
            pl.BlockSpec((1, D), lambda i: (0, 0)),
            pl.BlockSpec((D, D), lambda i: (0, 0)),
        ],
        out_specs=pl.BlockSpec((NC, BLK, HD), lambda i: (0, i, 0)),
        out_shape=jax.ShapeDtypeStruct((NC, N, HD), jnp.float32),
    )(z, y1, cnt, b1, W2)


def _tc3(z, y2, cnt, b2, batch3, Wout_p, bout_p):
    return pl.pallas_call(
        _tc3_body,
        grid=(NG,),
        in_specs=[
            pl.BlockSpec((NC, BLK, HD), lambda i: (0, i, 0)),
            pl.BlockSpec((NC, BLK, HD), lambda i: (0, i, 0)),
            pl.BlockSpec((NC, BLK, 16), lambda i: (0, i, 0)),
            pl.BlockSpec((1, D), lambda i: (0, 0)),
            pl.BlockSpec((1, 1, BLK), lambda i: (i, 0, 0)),
            pl.BlockSpec((D, D), lambda i: (0, 0)),
            pl.BlockSpec((1, D), lambda i: (0, 0)),
        ],
        out_specs=pl.BlockSpec((G, D), lambda i: (0, 0)),
        out_shape=jax.ShapeDtypeStruct((G, D), jnp.float32),
        scratch_shapes=[
            pltpu.VMEM((G, D), jnp.float32),
            pltpu.VMEM((G, D), jnp.float32),
        ],
    )(z, y2, cnt, b2, batch3, Wout_p, bout_p)


def kernel(x, edge_index, batch, W1, b1, W2, b2, W_out, b_out):
    src_r = edge_index[0].astype(jnp.int32).reshape(NS, NCH, CH)
    dst_r = edge_index[1].astype(jnp.int32).reshape(NS, NCH, CH)
    batch3 = batch.astype(jnp.int32).reshape(NG, 1, BLK)
    b1r = b1.reshape(1, D)
    b2r = b2.reshape(1, D)
    Wout_p = jnp.zeros((D, D), jnp.float32).at[:, :OUT_DIM].set(W_out)
    bout_p = jnp.zeros((1, D), jnp.float32).at[0, :OUT_DIM].set(b_out)

    xw1 = _tc_mm(x, W1)
    cnt = _deg_kernel(dst_r)
    y1 = _tc_scale(xw1, cnt)
    z1 = _edge_kernel(y1, src_r, dst_r)
    y2 = _tc2(z1, y1, cnt, b1r, W2)
    z2 = _edge_kernel(y2, src_r, dst_r)
    out = _tc3(z2, y2, cnt, b2r, batch3, Wout_p, bout_p)
    return out[:G, :OUT_DIM]


# speedup vs baseline: 1.3738x; 1.1145x over previous
import functools

import jax
import jax.numpy as jnp
from jax import lax
from jax.experimental import pallas as pl
from jax.experimental.pallas import tpu as pltpu
from jax.experimental.pallas import tpu_sc as plsc

N = 10000
E = 320000
D = 128
HD = D // 2
G = 64
OUT_DIM = 10

NC = 2
NS = 16
CH = 100
NCH = E // NS // CH
RZ = 624

_MESH = plsc.VectorSubcoreMesh(core_axis_name="c", subcore_axis_name="s",
                               num_cores=NC, num_subcores=NS)


@functools.partial(
    pl.kernel,
    out_type=jax.ShapeDtypeStruct((NC, N, 16), jnp.float32),
    mesh=_MESH,
    compiler_params=pltpu.CompilerParams(use_tc_tiling_on_sc=False),
    scratch_types=[
        pltpu.VMEM((NCH, CH), jnp.int32),
        pltpu.VMEM((CH, 16), jnp.float32),
        pltpu.VMEM_SHARED((N, 16), jnp.float32),
    ],
)
def _deg_kernel(dst_r, cnt_hbm, dst_v, ones_v, cnt_sh):
    core = lax.axis_index("c")
    sid = lax.axis_index("s")
    row0 = sid * RZ
    nblk = jnp.where(sid == NS - 1, (N - (NS - 1) * RZ) // 16, RZ // 16)

    zero = jnp.zeros((16,), jnp.float32)

    def zbody(i, _):
        ones_v[i, pl.ds(0, 16)] = zero
        return 0

    lax.fori_loop(0, 16, zbody, 0)

    def zdma(t, _):
        pltpu.sync_copy(ones_v.at[pl.ds(0, 16)],
                        cnt_sh.at[pl.ds(row0 + t * 16, 16)])
        return 0

    lax.fori_loop(0, nblk, zdma, 0)

    one = jnp.ones((16,), jnp.float32)

    def obody(i, _):
        ones_v[i, pl.ds(0, 16)] = one
        return 0

    lax.fori_loop(0, CH, obody, 0)

    pltpu.sync_copy(dst_r.at[sid], dst_v)
    plsc.subcore_barrier()

    j0 = core * (NCH // 2)

    def ebody(j, _):
        pltpu.sync_copy(ones_v, cnt_sh.at[dst_v.at[j0 + j]], add=True)
        return 0

    lax.fori_loop(0, NCH // 2, ebody, 0)

    plsc.subcore_barrier()

    def wdma(t, _):
        off = row0 + t * 16
        pltpu.sync_copy(cnt_sh.at[pl.ds(off, 16)],
                        cnt_hbm.at[core, pl.ds(off, 16)])
        return 0

    lax.fori_loop(0, nblk, wdma, 0)


@functools.partial(
    pl.kernel,
    out_type=jax.ShapeDtypeStruct((NC, N, HD), jnp.float32),
    mesh=_MESH,
    compiler_params=pltpu.CompilerParams(use_tc_tiling_on_sc=False),
    scratch_types=[
        pltpu.VMEM((NCH, CH), jnp.int32),
        pltpu.VMEM((NCH, CH), jnp.int32),
        pltpu.VMEM((2, CH, HD), jnp.float32),
        pltpu.VMEM_SHARED((N, HD), jnp.float32),
        pltpu.SemaphoreType.DMA,
        pltpu.SemaphoreType.DMA,
        pltpu.SemaphoreType.DMA,
        pltpu.SemaphoreType.DMA,
    ],
)
def _edge_kernel(y_hbm, src_r, dst_r, z_hbm, src_v, dst_v, rows_v, z_sh,
                 gs0, gs1, ss0, ss1):
    core = lax.axis_index("c")
    sid = lax.axis_index("s")
    row0 = sid * RZ

    ytab = y_hbm.at[core]

    @pl.when(sid < NS - 1)
    def _():
        pltpu.sync_copy(ytab.at[pl.ds(row0, RZ)], z_sh.at[pl.ds(row0, RZ)])

    @pl.when(sid == NS - 1)
    def _():
        last = N - (NS - 1) * RZ
        pltpu.sync_copy(ytab.at[pl.ds((NS - 1) * RZ, last)],
                        z_sh.at[pl.ds((NS - 1) * RZ, last)])

    pltpu.sync_copy(src_r.at[sid], src_v)
    pltpu.sync_copy(dst_r.at[sid], dst_v)
    plsc.subcore_barrier()

    pltpu.async_copy(ytab.at[src_v.at[0]], rows_v.at[0], gs0)

    def ebody(t, _):
        j0 = 2 * t
        j1 = j0 + 1
        pltpu.async_copy(ytab.at[src_v.at[j1]], rows_v.at[1], gs1)
        pltpu.make_async_copy(ytab.at[src_v.at[j0]], rows_v.at[0],
                              gs0).wait()
        pltpu.sync_copy(rows_v.at[0], z_sh.at[dst_v.at[j0]], add=True)

        @pl.when(t < NCH // 2 - 1)
        def _():
            pltpu.async_copy(ytab.at[src_v.at[j0 + 2]], rows_v.at[0], gs0)

        pltpu.make_async_copy(ytab.at[src_v.at[j1]], rows_v.at[1],
                              gs1).wait()
        pltpu.sync_copy(rows_v.at[1], z_sh.at[dst_v.at[j1]], add=True)
        return 0

    lax.fori_loop(0, NCH // 2, ebody, 0)

    plsc.subcore_barrier()

    @pl.when(sid < NS - 1)
    def _():
        pltpu.sync_copy(z_sh.at[pl.ds(row0, RZ)],
                        z_hbm.at[core, pl.ds(row0, RZ)])

    @pl.when(sid == NS - 1)
    def _():
        last = N - (NS - 1) * RZ
        pltpu.sync_copy(z_sh.at[pl.ds((NS - 1) * RZ, last)],
                        z_hbm.at[core, pl.ds((NS - 1) * RZ, last)])


BLK = 2000
NG = N // BLK


def _mm_body(x_ref, w_ref, o_ref):
    o_ref[...] = jnp.dot(x_ref[...], w_ref[...],
                         preferred_element_type=jnp.float32)


def _scale_body(xw_ref, cnt_ref, y_ref):
    s = jnp.sum(jnp.sum(cnt_ref[...], axis=2), axis=0) * 0.0625
    dinv = lax.rsqrt(s + 1.0)
    y = xw_ref[...] * dinv[:, None]
    y_ref[0] = y[:, :HD]
    y_ref[1] = y[:, HD:]


def _tc2_body(z_ref, cnt_ref, b1_ref, w2_ref, y2_ref):
    s = jnp.sum(jnp.sum(cnt_ref[...], axis=2), axis=0) * 0.0625
    dinv = lax.rsqrt(s + 1.0)
    zy = jnp.concatenate([z_ref[0], z_ref[1]], axis=1)
    agg = zy * dinv[:, None] + b1_ref[...]
    h = jnp.maximum(agg, 0.0)
    y2 = jnp.dot(h, w2_ref[...],
                 preferred_element_type=jnp.float32) * dinv[:, None]
    y2_ref[0] = y2[:, :HD]
    y2_ref[1] = y2[:, HD:]


def _tc3_body(z_ref, cnt_ref, b2_ref, batch_ref, wout_ref, bout_ref,
              o_ref, accs, accc):
    i = pl.program_id(0)

    @pl.when(i == 0)
    def _():
        accs[...] = jnp.zeros((G, D), jnp.float32)
        accc[...] = jnp.zeros((G, D), jnp.float32)

    s = jnp.sum(jnp.sum(cnt_ref[...], axis=2), axis=0) * 0.0625
    dinv = lax.rsqrt(s + 1.0)
    zy = jnp.concatenate([z_ref[0], z_ref[1]], axis=1)
    h2 = zy * dinv[:, None] + b2_ref[...]

    bt = batch_ref[0]
    gids = lax.broadcasted_iota(jnp.int32, (G, 1), 0)
    ohT = (gids == bt).astype(jnp.float32)
    accs[...] += jnp.dot(ohT, h2, preferred_element_type=jnp.float32)
    accc[...] += jnp.sum(ohT, axis=1, keepdims=True)

    @pl.when(i == NG - 1)
    def _():
        counts = accc[:, 0:1]
        pooled = accs[...] / jnp.maximum(counts, 1.0)
        logits = jnp.dot(pooled, wout_ref[...],
                         preferred_element_type=jnp.float32) + bout_ref[...]
        m = jnp.max(logits, axis=1, keepdims=True)
        e = jnp.exp(logits - m)
        ssum = jnp.sum(e, axis=1, keepdims=True)
        o_ref[...] = (logits - m) - jnp.log(ssum)


def _tc_mm(x, W1):
    return pl.pallas_call(
        _mm_body,
        grid=(NG,),
        in_specs=[
            pl.BlockSpec((BLK, D), lambda i: (i, 0)),
            pl.BlockSpec((D, D), lambda i: (0, 0)),
        ],
        out_specs=pl.BlockSpec((BLK, D), lambda i: (i, 0)),
        out_shape=jax.ShapeDtypeStruct((N, D), jnp.float32),
    )(x, W1)


def _tc_scale(xw, cnt):
    return pl.pallas_call(
        _scale_body,
        grid=(NG,),
        in_specs=[
            pl.BlockSpec((BLK, D), lambda i: (i, 0)),
            pl.BlockSpec((NC, BLK, 16), lambda i: (0, i, 0)),
        ],
        out_specs=pl.BlockSpec((NC, BLK, HD), lambda i: (0, i, 0)),
        out_shape=jax.ShapeDtypeStruct((NC, N, HD), jnp.float32),
    )(xw, cnt)


def _tc2(z, cnt, b1, W2):
    return pl.pallas_call(
        _tc2_body,
        grid=(NG,),
        in_specs=[
            pl.BlockSpec((NC, BLK, HD), lambda i: (0, i, 0)),
            pl.BlockSpec((NC, BLK, 16), lambda i: (0, i, 0)),
            pl.BlockSpec((1, D), lambda i: (0, 0)),
            pl.BlockSpec((D, D), lambda i: (0, 0)),
        ],
        out_specs=pl.BlockSpec((NC, BLK, HD), lambda i: (0, i, 0)),
        out_shape=jax.ShapeDtypeStruct((NC, N, HD), jnp.float32),
    )(z, cnt, b1, W2)


def _tc3(z, cnt, b2, batch3, W_out, b_out):
    return pl.pallas_call(
        _tc3_body,
        grid=(NG,),
        in_specs=[
            pl.BlockSpec((NC, BLK, HD), lambda i: (0, i, 0)),
            pl.BlockSpec((NC, BLK, 16), lambda i: (0, i, 0)),
            pl.BlockSpec((1, D), lambda i: (0, 0)),
            pl.BlockSpec((1, 1, BLK), lambda i: (i, 0, 0)),
            pl.BlockSpec((D, OUT_DIM), lambda i: (0, 0)),
            pl.BlockSpec((1, OUT_DIM), lambda i: (0, 0)),
        ],
        out_specs=pl.BlockSpec((G, OUT_DIM), lambda i: (0, 0)),
        out_shape=jax.ShapeDtypeStruct((G, OUT_DIM), jnp.float32),
        scratch_shapes=[
            pltpu.VMEM((G, D), jnp.float32),
            pltpu.VMEM((G, D), jnp.float32),
        ],
    )(z, cnt, b2, batch3, W_out, b_out)


def kernel(x, edge_index, batch, W1, b1, W2, b2, W_out, b_out):
    src_r = edge_index[0].astype(jnp.int32).reshape(NS, NCH, CH)
    dst_r = edge_index[1].astype(jnp.int32).reshape(NS, NCH, CH)
    batch3 = batch.astype(jnp.int32).reshape(NG, 1, BLK)
    b1r = b1.reshape(1, D)
    b2r = b2.reshape(1, D)
    boutr = b_out.reshape(1, OUT_DIM)

    xw1 = _tc_mm(x, W1)
    cnt = _deg_kernel(dst_r)
    y1 = _tc_scale(xw1, cnt)
    z1 = _edge_kernel(y1, src_r, dst_r)
    y2 = _tc2(z1, cnt, b1r, W2)
    z2 = _edge_kernel(y2, src_r, dst_r)
    return _tc3(z2, cnt, b2r, batch3, W_out, boutr)


# speedup vs baseline: 1.6553x; 1.2049x over previous
import functools

import jax
import jax.numpy as jnp
from jax import lax
from jax.experimental import pallas as pl
from jax.experimental.pallas import tpu as pltpu
from jax.experimental.pallas import tpu_sc as plsc

N = 10000
E = 320000
D = 128
HD = D // 2
G = 64
OUT_DIM = 10

NC = 2
NS = 16
CH = 250
NCH = E // NS // CH
RZ = 624

_MESH = plsc.VectorSubcoreMesh(core_axis_name="c", subcore_axis_name="s",
                               num_cores=NC, num_subcores=NS)


@functools.partial(
    pl.kernel,
    out_type=jax.ShapeDtypeStruct((NC, N, 16), jnp.float32),
    mesh=_MESH,
    compiler_params=pltpu.CompilerParams(use_tc_tiling_on_sc=False),
    scratch_types=[
        pltpu.VMEM((NCH, CH), jnp.int32),
        pltpu.VMEM((CH, 16), jnp.float32),
        pltpu.VMEM_SHARED((N, 16), jnp.float32),
    ],
)
def _deg_kernel(dst_r, cnt_hbm, dst_v, ones_v, cnt_sh):
    core = lax.axis_index("c")
    sid = lax.axis_index("s")
    row0 = sid * RZ
    nblk = jnp.where(sid == NS - 1, (N - (NS - 1) * RZ) // 16, RZ // 16)

    zero = jnp.zeros((16,), jnp.float32)

    def zbody(i, _):
        ones_v[i, pl.ds(0, 16)] = zero
        return 0

    lax.fori_loop(0, 16, zbody, 0)

    def zdma(t, _):
        pltpu.sync_copy(ones_v.at[pl.ds(0, 16)],
                        cnt_sh.at[pl.ds(row0 + t * 16, 16)])
        return 0

    lax.fori_loop(0, nblk, zdma, 0)

    one = jnp.ones((16,), jnp.float32)

    def obody(i, _):
        ones_v[i, pl.ds(0, 16)] = one
        return 0

    lax.fori_loop(0, CH, obody, 0)

    pltpu.sync_copy(dst_r.at[sid], dst_v)
    plsc.subcore_barrier()

    j0 = core * (NCH // 2)

    def ebody(j, _):
        pltpu.sync_copy(ones_v, cnt_sh.at[dst_v.at[j0 + j]], add=True)
        return 0

    lax.fori_loop(0, NCH // 2, ebody, 0)

    plsc.subcore_barrier()

    def wdma(t, _):
        off = row0 + t * 16
        pltpu.sync_copy(cnt_sh.at[pl.ds(off, 16)],
                        cnt_hbm.at[core, pl.ds(off, 16)])
        return 0

    lax.fori_loop(0, nblk, wdma, 0)


@functools.partial(
    pl.kernel,
    out_type=jax.ShapeDtypeStruct((NC, N, HD), jnp.float32),
    mesh=_MESH,
    compiler_params=pltpu.CompilerParams(use_tc_tiling_on_sc=False),
    scratch_types=[
        pltpu.VMEM((NCH, CH), jnp.int32),
        pltpu.VMEM((NCH, CH), jnp.int32),
        pltpu.VMEM((2, CH, HD), jnp.float32),
        pltpu.VMEM_SHARED((N, HD), jnp.float32),
        pltpu.SemaphoreType.DMA,
        pltpu.SemaphoreType.DMA,
        pltpu.SemaphoreType.DMA,
        pltpu.SemaphoreType.DMA,
    ],
)
def _edge_kernel(y_hbm, src_r, dst_r, z_hbm, src_v, dst_v, rows_v, z_sh,
                 gs0, gs1, ss0, ss1):
    core = lax.axis_index("c")
    sid = lax.axis_index("s")
    row0 = sid * RZ

    ytab = y_hbm.at[core]

    @pl.when(sid < NS - 1)
    def _():
        pltpu.sync_copy(ytab.at[pl.ds(row0, RZ)], z_sh.at[pl.ds(row0, RZ)])

    @pl.when(sid == NS - 1)
    def _():
        last = N - (NS - 1) * RZ
        pltpu.sync_copy(ytab.at[pl.ds((NS - 1) * RZ, last)],
                        z_sh.at[pl.ds((NS - 1) * RZ, last)])

    pltpu.sync_copy(src_r.at[sid], src_v)
    pltpu.sync_copy(dst_r.at[sid], dst_v)
    plsc.subcore_barrier()

    pltpu.async_copy(ytab.at[src_v.at[0]], rows_v.at[0], gs0)

    def ebody(t, _):
        j0 = 2 * t
        j1 = j0 + 1
        pltpu.async_copy(ytab.at[src_v.at[j1]], rows_v.at[1], gs1)
        pltpu.make_async_copy(ytab.at[src_v.at[j0]], rows_v.at[0],
                              gs0).wait()
        pltpu.sync_copy(rows_v.at[0], z_sh.at[dst_v.at[j0]], add=True)

        @pl.when(t < NCH // 2 - 1)
        def _():
            pltpu.async_copy(ytab.at[src_v.at[j0 + 2]], rows_v.at[0], gs0)

        pltpu.make_async_copy(ytab.at[src_v.at[j1]], rows_v.at[1],
                              gs1).wait()
        pltpu.sync_copy(rows_v.at[1], z_sh.at[dst_v.at[j1]], add=True)
        return 0

    lax.fori_loop(0, NCH // 2, ebody, 0)

    plsc.subcore_barrier()

    @pl.when(sid < NS - 1)
    def _():
        pltpu.sync_copy(z_sh.at[pl.ds(row0, RZ)],
                        z_hbm.at[core, pl.ds(row0, RZ)])

    @pl.when(sid == NS - 1)
    def _():
        last = N - (NS - 1) * RZ
        pltpu.sync_copy(z_sh.at[pl.ds((NS - 1) * RZ, last)],
                        z_hbm.at[core, pl.ds((NS - 1) * RZ, last)])


BLK = 2000
NG = N // BLK


def _mm_body(x_ref, w_ref, o_ref):
    o_ref[...] = jnp.dot(x_ref[...], w_ref[...],
                         preferred_element_type=jnp.float32)


def _scale_body(xw_ref, cnt_ref, y_ref):
    s = jnp.sum(jnp.sum(cnt_ref[...], axis=2), axis=0) * 0.0625
    dinv = lax.rsqrt(s + 1.0)
    y = xw_ref[...] * dinv[:, None]
    y_ref[0] = y[:, :HD]
    y_ref[1] = y[:, HD:]


def _tc2_body(z_ref, cnt_ref, b1_ref, w2_ref, y2_ref):
    s = jnp.sum(jnp.sum(cnt_ref[...], axis=2), axis=0) * 0.0625
    dinv = lax.rsqrt(s + 1.0)
    zy = jnp.concatenate([z_ref[0], z_ref[1]], axis=1)
    agg = zy * dinv[:, None] + b1_ref[...]
    h = jnp.maximum(agg, 0.0)
    y2 = jnp.dot(h, w2_ref[...],
                 preferred_element_type=jnp.float32) * dinv[:, None]
    y2_ref[0] = y2[:, :HD]
    y2_ref[1] = y2[:, HD:]


def _tc3_body(z_ref, cnt_ref, b2_ref, batch_ref, wout_ref, bout_ref,
              o_ref, accs, accc):
    i = pl.program_id(0)

    @pl.when(i == 0)
    def _():
        accs[...] = jnp.zeros((G, D), jnp.float32)
        accc[...] = jnp.zeros((G, D), jnp.float32)

    s = jnp.sum(jnp.sum(cnt_ref[...], axis=2), axis=0) * 0.0625
    dinv = lax.rsqrt(s + 1.0)
    zy = jnp.concatenate([z_ref[0], z_ref[1]], axis=1)
    h2 = zy * dinv[:, None] + b2_ref[...]

    bt = batch_ref[0]
    gids = lax.broadcasted_iota(jnp.int32, (G, 1), 0)
    ohT = (gids == bt).astype(jnp.float32)
    accs[...] += jnp.dot(ohT, h2, preferred_element_type=jnp.float32)
    accc[...] += jnp.sum(ohT, axis=1, keepdims=True)

    @pl.when(i == NG - 1)
    def _():
        counts = accc[:, 0:1]
        pooled = accs[...] / jnp.maximum(counts, 1.0)
        logits = jnp.dot(pooled, wout_ref[...],
                         preferred_element_type=jnp.float32) + bout_ref[...]
        m = jnp.max(logits, axis=1, keepdims=True)
        e = jnp.exp(logits - m)
        ssum = jnp.sum(e, axis=1, keepdims=True)
        o_ref[...] = (logits - m) - jnp.log(ssum)


def _tc_mm(x, W1):
    return pl.pallas_call(
        _mm_body,
        grid=(NG,),
        in_specs=[
            pl.BlockSpec((BLK, D), lambda i: (i, 0)),
            pl.BlockSpec((D, D), lambda i: (0, 0)),
        ],
        out_specs=pl.BlockSpec((BLK, D), lambda i: (i, 0)),
        out_shape=jax.ShapeDtypeStruct((N, D), jnp.float32),
    )(x, W1)


def _tc_scale(xw, cnt):
    return pl.pallas_call(
        _scale_body,
        grid=(NG,),
        in_specs=[
            pl.BlockSpec((BLK, D), lambda i: (i, 0)),
            pl.BlockSpec((NC, BLK, 16), lambda i: (0, i, 0)),
        ],
        out_specs=pl.BlockSpec((NC, BLK, HD), lambda i: (0, i, 0)),
        out_shape=jax.ShapeDtypeStruct((NC, N, HD), jnp.float32),
    )(xw, cnt)


def _tc2(z, cnt, b1, W2):
    return pl.pallas_call(
        _tc2_body,
        grid=(NG,),
        in_specs=[
            pl.BlockSpec((NC, BLK, HD), lambda i: (0, i, 0)),
            pl.BlockSpec((NC, BLK, 16), lambda i: (0, i, 0)),
            pl.BlockSpec((1, D), lambda i: (0, 0)),
            pl.BlockSpec((D, D), lambda i: (0, 0)),
        ],
        out_specs=pl.BlockSpec((NC, BLK, HD), lambda i: (0, i, 0)),
        out_shape=jax.ShapeDtypeStruct((NC, N, HD), jnp.float32),
    )(z, cnt, b1, W2)


def _tc3(z, cnt, b2, batch3, W_out, b_out):
    return pl.pallas_call(
        _tc3_body,
        grid=(NG,),
        in_specs=[
            pl.BlockSpec((NC, BLK, HD), lambda i: (0, i, 0)),
            pl.BlockSpec((NC, BLK, 16), lambda i: (0, i, 0)),
            pl.BlockSpec((1, D), lambda i: (0, 0)),
            pl.BlockSpec((1, 1, BLK), lambda i: (i, 0, 0)),
            pl.BlockSpec((D, OUT_DIM), lambda i: (0, 0)),
            pl.BlockSpec((1, OUT_DIM), lambda i: (0, 0)),
        ],
        out_specs=pl.BlockSpec((G, OUT_DIM), lambda i: (0, 0)),
        out_shape=jax.ShapeDtypeStruct((G, OUT_DIM), jnp.float32),
        scratch_shapes=[
            pltpu.VMEM((G, D), jnp.float32),
            pltpu.VMEM((G, D), jnp.float32),
        ],
    )(z, cnt, b2, batch3, W_out, b_out)


def kernel(x, edge_index, batch, W1, b1, W2, b2, W_out, b_out):
    src_r = edge_index[0].astype(jnp.int32).reshape(NS, NCH, CH)
    dst_r = edge_index[1].astype(jnp.int32).reshape(NS, NCH, CH)
    batch3 = batch.astype(jnp.int32).reshape(NG, 1, BLK)
    b1r = b1.reshape(1, D)
    b2r = b2.reshape(1, D)
    boutr = b_out.reshape(1, OUT_DIM)

    xw1 = _tc_mm(x, W1)
    cnt = _deg_kernel(dst_r)
    y1 = _tc_scale(xw1, cnt)
    z1 = _edge_kernel(y1, src_r, dst_r)
    y2 = _tc2(z1, cnt, b1r, W2)
    z2 = _edge_kernel(y2, src_r, dst_r)
    return _tc3(z2, cnt, b2r, batch3, W_out, boutr)


# speedup vs baseline: 1.6681x; 1.0077x over previous
import functools

import jax
import jax.numpy as jnp
from jax import lax
from jax.experimental import pallas as pl
from jax.experimental.pallas import tpu as pltpu
from jax.experimental.pallas import tpu_sc as plsc

N = 10000
E = 320000
D = 128
HD = D // 2
G = 64
OUT_DIM = 10

NC = 2
NS = 16
CH = 250
NCH = E // NS // CH
RZ = 624

_MESH = plsc.VectorSubcoreMesh(core_axis_name="c", subcore_axis_name="s",
                               num_cores=NC, num_subcores=NS)


@functools.partial(
    pl.kernel,
    out_type=jax.ShapeDtypeStruct((NC, N, 16), jnp.float32),
    mesh=_MESH,
    compiler_params=pltpu.CompilerParams(use_tc_tiling_on_sc=False),
    scratch_types=[
        pltpu.VMEM((NCH, CH), jnp.int32),
        pltpu.VMEM((CH, 16), jnp.float32),
        pltpu.VMEM_SHARED((N, 16), jnp.float32),
        pltpu.SemaphoreType.DMA,
    ],
)
def _deg_kernel(dst_r, cnt_hbm, dst_v, ones_v, cnt_sh, dsem):
    core = lax.axis_index("c")
    sid = lax.axis_index("s")
    row0 = sid * RZ
    nblk = jnp.where(sid == NS - 1, (N - (NS - 1) * RZ) // 16, RZ // 16)

    zero = jnp.zeros((16,), jnp.float32)

    def zbody(i, _):
        ones_v[i, pl.ds(0, 16)] = zero
        return 0

    lax.fori_loop(0, 16, zbody, 0)

    def zdma(t, _):
        pltpu.sync_copy(ones_v.at[pl.ds(0, 16)],
                        cnt_sh.at[pl.ds(row0 + t * 16, 16)])
        return 0

    lax.fori_loop(0, nblk, zdma, 0)

    one = jnp.ones((16,), jnp.float32)

    def obody(i, _):
        ones_v[i, pl.ds(0, 16)] = one
        return 0

    lax.fori_loop(0, CH, obody, 0)

    pltpu.sync_copy(dst_r.at[sid], dst_v)
    plsc.subcore_barrier()

    j0 = core * (NCH // 2)

    def ebody(j, _):
        pltpu.async_copy(ones_v, cnt_sh.at[dst_v.at[j0 + j]], dsem, add=True)
        return 0

    lax.fori_loop(0, NCH // 2, ebody, 0)

    def dbody(j, _):
        pltpu.make_async_copy(ones_v, cnt_sh.at[dst_v.at[j0 + j]],
                              dsem).wait()
        return 0

    lax.fori_loop(0, NCH // 2, dbody, 0)

    plsc.subcore_barrier()

    def wdma(t, _):
        off = row0 + t * 16
        pltpu.sync_copy(cnt_sh.at[pl.ds(off, 16)],
                        cnt_hbm.at[core, pl.ds(off, 16)])
        return 0

    lax.fori_loop(0, nblk, wdma, 0)


@functools.partial(
    pl.kernel,
    out_type=jax.ShapeDtypeStruct((NC, N, HD), jnp.float32),
    mesh=_MESH,
    compiler_params=pltpu.CompilerParams(use_tc_tiling_on_sc=False),
    scratch_types=[
        pltpu.VMEM((NCH, CH), jnp.int32),
        pltpu.VMEM((NCH, CH), jnp.int32),
        pltpu.VMEM((2, CH, HD), jnp.float32),
        pltpu.VMEM_SHARED((N, HD), jnp.float32),
        pltpu.SemaphoreType.DMA,
        pltpu.SemaphoreType.DMA,
        pltpu.SemaphoreType.DMA,
        pltpu.SemaphoreType.DMA,
    ],
)
def _edge_kernel(y_hbm, src_r, dst_r, z_hbm, src_v, dst_v, rows_v, z_sh,
                 gs0, gs1, ss0, ss1):
    core = lax.axis_index("c")
    sid = lax.axis_index("s")
    row0 = sid * RZ

    ytab = y_hbm.at[core]

    @pl.when(sid < NS - 1)
    def _():
        pltpu.sync_copy(ytab.at[pl.ds(row0, RZ)], z_sh.at[pl.ds(row0, RZ)])

    @pl.when(sid == NS - 1)
    def _():
        last = N - (NS - 1) * RZ
        pltpu.sync_copy(ytab.at[pl.ds((NS - 1) * RZ, last)],
                        z_sh.at[pl.ds((NS - 1) * RZ, last)])

    pltpu.sync_copy(src_r.at[sid], src_v)
    pltpu.sync_copy(dst_r.at[sid], dst_v)
    plsc.subcore_barrier()

    pltpu.async_copy(ytab.at[src_v.at[0]], rows_v.at[0], gs0)

    def ebody(t, _):
        j0 = 2 * t
        j1 = j0 + 1
        pltpu.async_copy(ytab.at[src_v.at[j1]], rows_v.at[1], gs1)
        pltpu.make_async_copy(ytab.at[src_v.at[j0]], rows_v.at[0],
                              gs0).wait()
        pltpu.sync_copy(rows_v.at[0], z_sh.at[dst_v.at[j0]], add=True)

        @pl.when(t < NCH // 2 - 1)
        def _():
            pltpu.async_copy(ytab.at[src_v.at[j0 + 2]], rows_v.at[0], gs0)

        pltpu.make_async_copy(ytab.at[src_v.at[j1]], rows_v.at[1],
                              gs1).wait()
        pltpu.sync_copy(rows_v.at[1], z_sh.at[dst_v.at[j1]], add=True)
        return 0

    lax.fori_loop(0, NCH // 2, ebody, 0)

    plsc.subcore_barrier()

    @pl.when(sid < NS - 1)
    def _():
        pltpu.sync_copy(z_sh.at[pl.ds(row0, RZ)],
                        z_hbm.at[core, pl.ds(row0, RZ)])

    @pl.when(sid == NS - 1)
    def _():
        last = N - (NS - 1) * RZ
        pltpu.sync_copy(z_sh.at[pl.ds((NS - 1) * RZ, last)],
                        z_hbm.at[core, pl.ds((NS - 1) * RZ, last)])


BLK = 2000
NG = N // BLK


def _tc1_body(x_ref, w_ref, cnt_ref, y_ref):
    s = jnp.sum(jnp.sum(cnt_ref[...], axis=2), axis=0) * 0.0625
    dinv = lax.rsqrt(s + 1.0)
    xw = jnp.dot(x_ref[...], w_ref[...], preferred_element_type=jnp.float32)
    y = xw * dinv[:, None]
    y_ref[0] = y[:, :HD]
    y_ref[1] = y[:, HD:]


def _tc2_body(z_ref, cnt_ref, b1_ref, w2_ref, y2_ref):
    s = jnp.sum(jnp.sum(cnt_ref[...], axis=2), axis=0) * 0.0625
    dinv = lax.rsqrt(s + 1.0)
    zy = jnp.concatenate([z_ref[0], z_ref[1]], axis=1)
    agg = zy * dinv[:, None] + b1_ref[...]
    h = jnp.maximum(agg, 0.0)
    y2 = jnp.dot(h, w2_ref[...],
                 preferred_element_type=jnp.float32) * dinv[:, None]
    y2_ref[0] = y2[:, :HD]
    y2_ref[1] = y2[:, HD:]


def _tc3_body(z_ref, cnt_ref, b2_ref, batch_ref, wout_ref, bout_ref,
              o_ref, accs, accc):
    i = pl.program_id(0)

    @pl.when(i == 0)
    def _():
        accs[...] = jnp.zeros((G, D), jnp.float32)
        accc[...] = jnp.zeros((G, D), jnp.float32)

    s = jnp.sum(jnp.sum(cnt_ref[...], axis=2), axis=0) * 0.0625
    dinv = lax.rsqrt(s + 1.0)
    zy = jnp.concatenate([z_ref[0], z_ref[1]], axis=1)
    h2 = zy * dinv[:, None] + b2_ref[...]

    bt = batch_ref[0]
    gids = lax.broadcasted_iota(jnp.int32, (G, 1), 0)
    ohT = (gids == bt).astype(jnp.float32)
    accs[...] += jnp.dot(ohT, h2, preferred_element_type=jnp.float32)
    accc[...] += jnp.sum(ohT, axis=1, keepdims=True)

    @pl.when(i == NG - 1)
    def _():
        counts = accc[:, 0:1]
        pooled = accs[...] / jnp.maximum(counts, 1.0)
        logits = jnp.dot(pooled, wout_ref[...],
                         preferred_element_type=jnp.float32) + bout_ref[...]
        m = jnp.max(logits, axis=1, keepdims=True)
        e = jnp.exp(logits - m)
        ssum = jnp.sum(e, axis=1, keepdims=True)
        o_ref[...] = (logits - m) - jnp.log(ssum)


def _tc1(x, W1, cnt):
    return pl.pallas_call(
        _tc1_body,
        grid=(NG,),
        in_specs=[
            pl.BlockSpec((BLK, D), lambda i: (i, 0)),
            pl.BlockSpec((D, D), lambda i: (0, 0)),
            pl.BlockSpec((NC, BLK, 16), lambda i: (0, i, 0)),
        ],
        out_specs=pl.BlockSpec((NC, BLK, HD), lambda i: (0, i, 0)),
        out_shape=jax.ShapeDtypeStruct((NC, N, HD), jnp.float32),
    )(x, W1, cnt)


def _tc2(z, cnt, b1, W2):
    return pl.pallas_call(
        _tc2_body,
        grid=(NG,),
        in_specs=[
            pl.BlockSpec((NC, BLK, HD), lambda i: (0, i, 0)),
            pl.BlockSpec((NC, BLK, 16), lambda i: (0, i, 0)),
            pl.BlockSpec((1, D), lambda i: (0, 0)),
            pl.BlockSpec((D, D), lambda i: (0, 0)),
        ],
        out_specs=pl.BlockSpec((NC, BLK, HD), lambda i: (0, i, 0)),
        out_shape=jax.ShapeDtypeStruct((NC, N, HD), jnp.float32),
    )(z, cnt, b1, W2)


def _tc3(z, cnt, b2, batch3, W_out, b_out):
    return pl.pallas_call(
        _tc3_body,
        grid=(NG,),
        in_specs=[
            pl.BlockSpec((NC, BLK, HD), lambda i: (0, i, 0)),
            pl.BlockSpec((NC, BLK, 16), lambda i: (0, i, 0)),
            pl.BlockSpec((1, D), lambda i: (0, 0)),
            pl.BlockSpec((1, 1, BLK), lambda i: (i, 0, 0)),
            pl.BlockSpec((D, OUT_DIM), lambda i: (0, 0)),
            pl.BlockSpec((1, OUT_DIM), lambda i: (0, 0)),
        ],
        out_specs=pl.BlockSpec((G, OUT_DIM), lambda i: (0, 0)),
        out_shape=jax.ShapeDtypeStruct((G, OUT_DIM), jnp.float32),
        scratch_shapes=[
            pltpu.VMEM((G, D), jnp.float32),
            pltpu.VMEM((G, D), jnp.float32),
        ],
    )(z, cnt, b2, batch3, W_out, b_out)


def kernel(x, edge_index, batch, W1, b1, W2, b2, W_out, b_out):
    src_r = edge_index[0].astype(jnp.int32).reshape(NS, NCH, CH)
    dst_r = edge_index[1].astype(jnp.int32).reshape(NS, NCH, CH)
    batch3 = batch.astype(jnp.int32).reshape(NG, 1, BLK)
    b1r = b1.reshape(1, D)
    b2r = b2.reshape(1, D)
    boutr = b_out.reshape(1, OUT_DIM)

    cnt = _deg_kernel(dst_r)
    y1 = _tc1(x, W1, cnt)
    z1 = _edge_kernel(y1, src_r, dst_r)
    y2 = _tc2(z1, cnt, b1r, W2)
    z2 = _edge_kernel(y2, src_r, dst_r)
    return _tc3(z2, cnt, b2r, batch3, W_out, boutr)
